# Initial kernel scaffold; baseline (speedup 1.0000x reference)
import jax
import jax.numpy as jnp
from jax.experimental import pallas as pl


def kernel(x, W_enc, W_dec, b_pre):
    raise NotImplementedError("write your pallas kernel here")


# speedup vs baseline: 3.5392x; 3.5392x over previous
import functools

import jax
import jax.numpy as jnp
from jax import lax
from jax.experimental import pallas as pl
from jax.experimental.pallas import tpu as pltpu

K = 32


def _encode_body(n_blocks, xb_ref, wb_ref, bp_ref, hs_ref, cnt_ref, h_acc):
    j = pl.program_id(1)
    bn = wb_ref.shape[1]
    xb = xb_ref[...] - bp_ref[...]
    hblk = jnp.dot(xb, wb_ref[...], preferred_element_type=jnp.float32)
    hblk = jnp.maximum(hblk, 0.0)
    h_acc[:, pl.ds(j * bn, bn)] = hblk

    @pl.when(j == n_blocks - 1)
    def _():
        h = h_acc[...]
        br = h.shape[0]
        lo = jnp.zeros((br, 1), jnp.int32)
        hi = jnp.full((br, 1), 0x7F800000, jnp.int32)

        def body(_, carry):
            lo, hi = carry
            mid = lo + lax.div(hi - lo, 2)
            midf = lax.bitcast_convert_type(mid, jnp.float32)
            cnt = jnp.sum((h >= midf).astype(jnp.float32), axis=1,
                          keepdims=True)
            ge = cnt >= float(K)
            return (jnp.where(ge, mid, lo), jnp.where(ge, hi, mid))

        lo, hi = lax.fori_loop(0, 31, body, (lo, hi))
        t = lax.bitcast_convert_type(lo, jnp.float32)
        hs = jnp.where(h >= t, h, 0.0)
        hs_ref[...] = hs
        cnt_ref[...] = jnp.sum((hs > 0.0).astype(jnp.float32), axis=1,
                               keepdims=True)


def _decode_body(k_blocks, hs_ref, wd_ref, x_ref, bp_ref, xhat_ref, res_ref,
                 acc):
    k = pl.program_id(1)

    @pl.when(k == 0)
    def _():
        acc[...] = jnp.zeros_like(acc)

    acc[...] += jnp.dot(hs_ref[...], wd_ref[...],
                        preferred_element_type=jnp.float32)

    @pl.when(k == k_blocks - 1)
    def _():
        xh = acc[...] + bp_ref[...]
        xhat_ref[...] = xh
        d = x_ref[...] - xh
        res_ref[...] = jnp.broadcast_to(jnp.sum(d * d), (1, 1, 128))


def kernel(x, W_enc, W_dec, b_pre):
    B, D = x.shape
    N = W_enc.shape[1]
    bp2 = b_pre.reshape(1, D)

    BR = min(128, B)
    BN = min(1024, N)
    rb, nb = B // BR, N // BN
    hs, cnt = pl.pallas_call(
        functools.partial(_encode_body, nb),
        grid=(rb, nb),
        in_specs=[
            pl.BlockSpec((BR, D), lambda r, n: (r, 0)),
            pl.BlockSpec((D, BN), lambda r, n: (0, n)),
            pl.BlockSpec((1, D), lambda r, n: (0, 0)),
        ],
        out_specs=[
            pl.BlockSpec((BR, N), lambda r, n: (r, 0)),
            pl.BlockSpec((BR, 1), lambda r, n: (r, 0)),
        ],
        out_shape=[
            jax.ShapeDtypeStruct((B, N), jnp.float32),
            jax.ShapeDtypeStruct((B, 1), jnp.float32),
        ],
        scratch_shapes=[pltpu.VMEM((BR, N), jnp.float32)],
        compiler_params=pltpu.CompilerParams(
            dimension_semantics=("parallel", "arbitrary")),
    )(x, W_enc, bp2)

    BR2 = min(256, B)
    BK = min(2048, N)
    rb2, kb = B // BR2, N // BK
    xhat, res = pl.pallas_call(
        functools.partial(_decode_body, kb),
        grid=(rb2, kb),
        in_specs=[
            pl.BlockSpec((BR2, BK), lambda r, k: (r, k)),
            pl.BlockSpec((BK, D), lambda r, k: (k, 0)),
            pl.BlockSpec((BR2, D), lambda r, k: (r, 0)),
            pl.BlockSpec((1, D), lambda r, k: (0, 0)),
        ],
        out_specs=[
            pl.BlockSpec((BR2, D), lambda r, k: (r, 0)),
            pl.BlockSpec((1, 1, 128), lambda r, k: (r, 0, 0)),
        ],
        out_shape=[
            jax.ShapeDtypeStruct((B, D), jnp.float32),
            jax.ShapeDtypeStruct((rb2, 1, 128), jnp.float32),
        ],
        scratch_shapes=[pltpu.VMEM((BR2, D), jnp.float32)],
        compiler_params=pltpu.CompilerParams(
            dimension_semantics=("parallel", "arbitrary")),
    )(hs, W_dec, x, bp2)

    recon_loss = jnp.sum(res[:, 0, 0]) / (B * D)
    l0 = jnp.sum(cnt) / B
    return (xhat, hs, recon_loss, l0)


# speedup vs baseline: 3.7244x; 1.0523x over previous
import functools

import jax
import jax.numpy as jnp
from jax import lax
from jax.experimental import pallas as pl
from jax.experimental.pallas import tpu as pltpu

K = 32


def _encode_body(n_blocks, xb_ref, wb_ref, bp_ref, hs_ref, cnt_ref, h_acc):
    j = pl.program_id(1)
    bn = wb_ref.shape[1]
    xb = xb_ref[...] - bp_ref[...]
    hblk = jnp.dot(xb, wb_ref[...], preferred_element_type=jnp.float32)
    hblk = jnp.maximum(hblk, 0.0)
    h_acc[:, pl.ds(j * bn, bn)] = hblk

    @pl.when(j == n_blocks - 1)
    def _():
        h = h_acc[...]
        br, n = h.shape
        m = jnp.max(h.reshape(br, K, n // K), axis=2)
        lo = lax.bitcast_convert_type(jnp.min(m, axis=1, keepdims=True),
                                      jnp.int32)
        hi = lax.bitcast_convert_type(jnp.max(m, axis=1, keepdims=True),
                                      jnp.int32) + 1

        def cond(carry):
            lo, hi = carry
            return jnp.any(hi - lo > 1)

        def body(carry):
            lo, hi = carry
            mid = lo + lax.div(hi - lo, 2)
            midf = lax.bitcast_convert_type(mid, jnp.float32)
            cnt = jnp.sum((h >= midf).astype(jnp.float32), axis=1,
                          keepdims=True)
            ge = cnt >= float(K)
            return (jnp.where(ge, mid, lo), jnp.where(ge, hi, mid))

        lo, hi = lax.while_loop(cond, body, (lo, hi))
        t = lax.bitcast_convert_type(lo, jnp.float32)
        hs = jnp.where(h >= t, h, 0.0)
        hs_ref[...] = hs
        cnt_ref[...] = jnp.sum((hs > 0.0).astype(jnp.float32), axis=1,
                               keepdims=True)


def _decode_body(k_blocks, hs_ref, wd_ref, x_ref, bp_ref, xhat_ref, res_ref,
                 acc):
    k = pl.program_id(1)

    @pl.when(k == 0)
    def _():
        acc[...] = jnp.zeros_like(acc)

    acc[...] += jnp.dot(hs_ref[...], wd_ref[...],
                        preferred_element_type=jnp.float32)

    @pl.when(k == k_blocks - 1)
    def _():
        xh = acc[...] + bp_ref[...]
        xhat_ref[...] = xh
        d = x_ref[...] - xh
        res_ref[...] = jnp.broadcast_to(jnp.sum(d * d), (1, 1, 128))


def kernel(x, W_enc, W_dec, b_pre):
    B, D = x.shape
    N = W_enc.shape[1]
    bp2 = b_pre.reshape(1, D)

    BR = min(128, B)
    BN = min(1024, N)
    rb, nb = B // BR, N // BN
    hs, cnt = pl.pallas_call(
        functools.partial(_encode_body, nb),
        grid=(rb, nb),
        in_specs=[
            pl.BlockSpec((BR, D), lambda r, n: (r, 0)),
            pl.BlockSpec((D, BN), lambda r, n: (0, n)),
            pl.BlockSpec((1, D), lambda r, n: (0, 0)),
        ],
        out_specs=[
            pl.BlockSpec((BR, N), lambda r, n: (r, 0)),
            pl.BlockSpec((BR, 1), lambda r, n: (r, 0)),
        ],
        out_shape=[
            jax.ShapeDtypeStruct((B, N), jnp.float32),
            jax.ShapeDtypeStruct((B, 1), jnp.float32),
        ],
        scratch_shapes=[pltpu.VMEM((BR, N), jnp.float32)],
        compiler_params=pltpu.CompilerParams(
            dimension_semantics=("parallel", "arbitrary")),
    )(x, W_enc, bp2)

    BR2 = min(256, B)
    BK = min(2048, N)
    rb2, kb = B // BR2, N // BK
    xhat, res = pl.pallas_call(
        functools.partial(_decode_body, kb),
        grid=(rb2, kb),
        in_specs=[
            pl.BlockSpec((BR2, BK), lambda r, k: (r, k)),
            pl.BlockSpec((BK, D), lambda r, k: (k, 0)),
            pl.BlockSpec((BR2, D), lambda r, k: (r, 0)),
            pl.BlockSpec((1, D), lambda r, k: (0, 0)),
        ],
        out_specs=[
            pl.BlockSpec((BR2, D), lambda r, k: (r, 0)),
            pl.BlockSpec((1, 1, 128), lambda r, k: (r, 0, 0)),
        ],
        out_shape=[
            jax.ShapeDtypeStruct((B, D), jnp.float32),
            jax.ShapeDtypeStruct((rb2, 1, 128), jnp.float32),
        ],
        scratch_shapes=[pltpu.VMEM((BR2, D), jnp.float32)],
        compiler_params=pltpu.CompilerParams(
            dimension_semantics=("parallel", "arbitrary")),
    )(hs, W_dec, x, bp2)

    recon_loss = jnp.sum(res[:, 0, 0]) / (B * D)
    l0 = jnp.sum(cnt) / B
    return (xhat, hs, recon_loss, l0)


# speedup vs baseline: 4.2182x; 1.1326x over previous
import functools

import jax
import jax.numpy as jnp
from jax import lax
from jax.experimental import pallas as pl
from jax.experimental.pallas import tpu as pltpu

K = 32


def _encode_body(n_blocks, xb_ref, wb_ref, bp_ref, hs_ref, cnt_ref, h_acc):
    j = pl.program_id(1)
    bn = wb_ref.shape[1]
    xb = xb_ref[...] - bp_ref[...]
    hblk = jnp.dot(xb, wb_ref[...], preferred_element_type=jnp.float32)
    hblk = jnp.maximum(hblk, 0.0)
    h_acc[:, pl.ds(j * bn, bn)] = hblk

    @pl.when(j == n_blocks - 1)
    def _():
        h = h_acc[...]
        br, n = h.shape
        m = jnp.max(h.reshape(br, K, n // K), axis=2)
        lo = lax.bitcast_convert_type(jnp.min(m, axis=1, keepdims=True),
                                      jnp.int32)
        hi = lax.bitcast_convert_type(jnp.max(m, axis=1, keepdims=True),
                                      jnp.int32) + 1

        def cond(carry):
            lo, hi = carry
            return jnp.any(hi - lo > 1)

        def body(carry):
            lo, hi = carry
            mid = lo + lax.div(hi - lo, 2)
            midf = lax.bitcast_convert_type(mid, jnp.float32)
            cnt = jnp.sum((h >= midf).astype(jnp.float32), axis=1,
                          keepdims=True)
            ge = cnt >= float(K)
            return (jnp.where(ge, mid, lo), jnp.where(ge, hi, mid))

        lo, hi = lax.while_loop(cond, body, (lo, hi))
        t = lax.bitcast_convert_type(lo, jnp.float32)
        hs = jnp.where(h >= t, h, 0.0)
        hs_ref[...] = hs
        cnt_ref[...] = jnp.sum((hs > 0.0).astype(jnp.float32), axis=1,
                               keepdims=True)


def _decode_body(k_blocks, hs_ref, wd_ref, x_ref, bp_ref, xhat_ref, res_ref,
                 acc):
    k = pl.program_id(1)

    @pl.when(k == 0)
    def _():
        acc[...] = jnp.zeros_like(acc)

    acc[...] += jnp.dot(hs_ref[...].astype(jnp.bfloat16), wd_ref[...],
                        preferred_element_type=jnp.float32)

    @pl.when(k == k_blocks - 1)
    def _():
        xh = acc[...] + bp_ref[...]
        xhat_ref[...] = xh
        d = x_ref[...] - xh
        res_ref[...] = jnp.broadcast_to(jnp.sum(d * d), (1, 1, 128))


def kernel(x, W_enc, W_dec, b_pre):
    B, D = x.shape
    N = W_enc.shape[1]
    bp2 = b_pre.reshape(1, D)

    BR = min(128, B)
    BN = min(1024, N)
    rb, nb = B // BR, N // BN
    hs, cnt = pl.pallas_call(
        functools.partial(_encode_body, nb),
        grid=(rb, nb),
        in_specs=[
            pl.BlockSpec((BR, D), lambda r, n: (r, 0)),
            pl.BlockSpec((D, BN), lambda r, n: (0, n)),
            pl.BlockSpec((1, D), lambda r, n: (0, 0)),
        ],
        out_specs=[
            pl.BlockSpec((BR, N), lambda r, n: (r, 0)),
            pl.BlockSpec((BR, 1), lambda r, n: (r, 0)),
        ],
        out_shape=[
            jax.ShapeDtypeStruct((B, N), jnp.float32),
            jax.ShapeDtypeStruct((B, 1), jnp.float32),
        ],
        scratch_shapes=[pltpu.VMEM((BR, N), jnp.float32)],
        compiler_params=pltpu.CompilerParams(
            dimension_semantics=("parallel", "arbitrary")),
    )(x, W_enc, bp2)

    BR2 = min(512, B)
    BK = min(2048, N)
    rb2, kb = B // BR2, N // BK
    wd16 = W_dec.astype(jnp.bfloat16)
    xhat, res = pl.pallas_call(
        functools.partial(_decode_body, kb),
        grid=(rb2, kb),
        in_specs=[
            pl.BlockSpec((BR2, BK), lambda r, k: (r, k)),
            pl.BlockSpec((BK, D), lambda r, k: (k, 0)),
            pl.BlockSpec((BR2, D), lambda r, k: (r, 0)),
            pl.BlockSpec((1, D), lambda r, k: (0, 0)),
        ],
        out_specs=[
            pl.BlockSpec((BR2, D), lambda r, k: (r, 0)),
            pl.BlockSpec((1, 1, 128), lambda r, k: (r, 0, 0)),
        ],
        out_shape=[
            jax.ShapeDtypeStruct((B, D), jnp.float32),
            jax.ShapeDtypeStruct((rb2, 1, 128), jnp.float32),
        ],
        scratch_shapes=[pltpu.VMEM((BR2, D), jnp.float32)],
        compiler_params=pltpu.CompilerParams(
            dimension_semantics=("parallel", "arbitrary")),
    )(hs, wd16, x, bp2)

    recon_loss = jnp.sum(res[:, 0, 0]) / (B * D)
    l0 = jnp.sum(cnt) / B
    return (xhat, hs, recon_loss, l0)


# speedup vs baseline: 4.3699x; 1.0360x over previous
import functools

import jax
import jax.numpy as jnp
from jax import lax
from jax.experimental import pallas as pl
from jax.experimental.pallas import tpu as pltpu

K = 32


def _encode_body(n_blocks, xb_ref, wb_ref, bp_ref, hs_ref, cnt_ref, h_acc):
    j = pl.program_id(1)
    bn = wb_ref.shape[1]
    xb = (xb_ref[...] - bp_ref[...]).astype(jnp.bfloat16)
    hblk = jnp.dot(xb, wb_ref[...], preferred_element_type=jnp.float32)
    hblk = jnp.maximum(hblk, 0.0)
    h_acc[:, pl.ds(j * bn, bn)] = hblk

    @pl.when(j == n_blocks - 1)
    def _():
        h = h_acc[...]
        br, n = h.shape
        m = jnp.max(h.reshape(br, K, n // K), axis=2)
        lo = lax.bitcast_convert_type(jnp.min(m, axis=1, keepdims=True),
                                      jnp.int32)
        hi = lax.bitcast_convert_type(jnp.max(m, axis=1, keepdims=True),
                                      jnp.int32) + 1

        def cond(carry):
            lo, hi = carry
            return jnp.any(hi - lo > 1)

        def body(carry):
            lo, hi = carry
            mid = lo + lax.div(hi - lo, 2)
            midf = lax.bitcast_convert_type(mid, jnp.float32)
            cnt = jnp.sum((h >= midf).astype(jnp.float32), axis=1,
                          keepdims=True)
            ge = cnt >= float(K)
            return (jnp.where(ge, mid, lo), jnp.where(ge, hi, mid))

        lo, hi = lax.while_loop(cond, body, (lo, hi))
        t = lax.bitcast_convert_type(lo, jnp.float32)
        hs = jnp.where(h >= t, h, 0.0)
        hs_ref[...] = hs
        cnt_ref[...] = jnp.sum((hs > 0.0).astype(jnp.float32), axis=1,
                               keepdims=True)


def _decode_body(k_blocks, hs_ref, wd_ref, x_ref, bp_ref, xhat_ref, res_ref,
                 acc):
    k = pl.program_id(1)

    @pl.when(k == 0)
    def _():
        acc[...] = jnp.zeros_like(acc)

    acc[...] += jnp.dot(hs_ref[...].astype(jnp.bfloat16), wd_ref[...],
                        preferred_element_type=jnp.float32)

    @pl.when(k == k_blocks - 1)
    def _():
        xh = acc[...] + bp_ref[...]
        xhat_ref[...] = xh
        d = x_ref[...] - xh
        res_ref[...] = jnp.broadcast_to(jnp.sum(d * d), (1, 1, 128))


def kernel(x, W_enc, W_dec, b_pre):
    B, D = x.shape
    N = W_enc.shape[1]
    bp2 = b_pre.reshape(1, D)

    BR = min(128, B)
    BN = min(512, N)
    rb, nb = B // BR, N // BN
    we16 = W_enc.astype(jnp.bfloat16)
    hs, cnt = pl.pallas_call(
        functools.partial(_encode_body, nb),
        grid=(rb, nb),
        in_specs=[
            pl.BlockSpec((BR, D), lambda r, n: (r, 0)),
            pl.BlockSpec((D, BN), lambda r, n: (0, n)),
            pl.BlockSpec((1, D), lambda r, n: (0, 0)),
        ],
        out_specs=[
            pl.BlockSpec((BR, N), lambda r, n: (r, 0)),
            pl.BlockSpec((BR, 1), lambda r, n: (r, 0)),
        ],
        out_shape=[
            jax.ShapeDtypeStruct((B, N), jnp.float32),
            jax.ShapeDtypeStruct((B, 1), jnp.float32),
        ],
        scratch_shapes=[pltpu.VMEM((BR, N), jnp.float32)],
        compiler_params=pltpu.CompilerParams(
            dimension_semantics=("parallel", "arbitrary")),
    )(x, we16, bp2)

    BR2 = min(512, B)
    BK = min(2048, N)
    rb2, kb = B // BR2, N // BK
    wd16 = W_dec.astype(jnp.bfloat16)
    xhat, res = pl.pallas_call(
        functools.partial(_decode_body, kb),
        grid=(rb2, kb),
        in_specs=[
            pl.BlockSpec((BR2, BK), lambda r, k: (r, k)),
            pl.BlockSpec((BK, D), lambda r, k: (k, 0)),
            pl.BlockSpec((BR2, D), lambda r, k: (r, 0)),
            pl.BlockSpec((1, D), lambda r, k: (0, 0)),
        ],
        out_specs=[
            pl.BlockSpec((BR2, D), lambda r, k: (r, 0)),
            pl.BlockSpec((1, 1, 128), lambda r, k: (r, 0, 0)),
        ],
        out_shape=[
            jax.ShapeDtypeStruct((B, D), jnp.float32),
            jax.ShapeDtypeStruct((rb2, 1, 128), jnp.float32),
        ],
        scratch_shapes=[pltpu.VMEM((BR2, D), jnp.float32)],
        compiler_params=pltpu.CompilerParams(
            dimension_semantics=("parallel", "arbitrary")),
    )(hs, wd16, x, bp2)

    recon_loss = jnp.sum(res[:, 0, 0]) / (B * D)
    l0 = jnp.sum(cnt) / B
    return (xhat, hs, recon_loss, l0)
